# baseline (speedup 1.0000x reference)
import functools

import jax
import jax.numpy as jnp
from jax import lax
from jax.experimental import pallas as pl
from jax.experimental.pallas import tpu as pltpu
from jax.experimental.pallas import tpu_sc as plsc

NUM_CORES = 2
NUM_SUBCORES = 16
NUM_WORKERS = NUM_CORES * NUM_SUBCORES
LANES = 16


@functools.lru_cache(maxsize=None)
def _build_gather(batch, seq, embed_dim):
    assert embed_dim % 8 == 0 and batch % 128 == 0
    n_et = embed_dim // 8
    n_bt = batch // 128
    assert batch % NUM_WORKERS == 0
    bpw = batch // NUM_WORKERS
    assert bpw % 128 == 0
    tb_per_w = bpw // 128
    assert seq % 2 == 0

    mesh = plsc.VectorSubcoreMesh(
        core_axis_name="c",
        subcore_axis_name="s",
        num_cores=NUM_CORES,
        num_subcores=NUM_SUBCORES,
    )

    @functools.partial(
        pl.kernel,
        out_type=jax.ShapeDtypeStruct((seq, n_et, n_bt, 8, 128), jnp.float32),
        mesh=mesh,
        compiler_params=pltpu.CompilerParams(
            use_tc_tiling_on_sc=False, needs_layout_passes=False
        ),
        scratch_types=[
            pltpu.VMEM((bpw, embed_dim), jnp.float32),
            pltpu.VMEM((bpw, embed_dim), jnp.float32),
            pltpu.VMEM((n_et, tb_per_w, 8, 128), jnp.float32),
            pltpu.VMEM((bpw,), jnp.int32),
            pltpu.VMEM((bpw,), jnp.int32),
            pltpu.SemaphoreType.DMA,
            pltpu.SemaphoreType.DMA,
            pltpu.SemaphoreType.DMA,
            pltpu.SemaphoreType.DMA,
            pltpu.SemaphoreType.DMA,
        ],
    )
    def gather_kernel(
        table_hbm, capt_hbm, out_hbm,
        rows0, rows1, tbuf, idx0, idx1,
        g0, g1, i0, i1, wsem,
    ):
        wid = lax.axis_index("s") * NUM_CORES + lax.axis_index("c")
        b0 = wid * bpw
        tb0 = wid * tb_per_w
        rows = (rows0, rows1)
        idxs = (idx0, idx1)
        gsem = (g0, g1)
        isem = (i0, i1)
        iota = lax.iota(jnp.int32, LANES)

        def idx_slice(s):
            return capt_hbm.at[pl.ds(s * batch + b0, bpw)]

        def start_idx(s, b):
            pltpu.async_copy(idx_slice(s), idxs[b], isem[b])

        def wait_idx(s, b):
            pltpu.make_async_copy(idx_slice(s), idxs[b], isem[b]).wait()

        def start_gather(b):
            pltpu.async_copy(table_hbm.at[idxs[b]], rows[b], gsem[b])

        def wait_gather(b):
            pltpu.make_async_copy(table_hbm.at[idxs[b]], rows[b], gsem[b]).wait()

        def transpose(b):
            src = rows[b]

            @pl.loop(0, n_et)
            def _(te):
                col_base = te * 8
                for tb in range(tb_per_w):
                    for e_in in range(8):
                        col = jnp.full((LANES,), col_base + e_in, jnp.int32)
                        for bg in range(128 // LANES):
                            rvec = iota + (tb * 128 + bg * LANES)
                            vals = plsc.load_gather(src, [rvec, col])
                            tbuf[te, tb, e_in, pl.ds(bg * LANES, LANES)] = vals

        def start_writes(s):
            for te in range(n_et):
                pltpu.async_copy(
                    tbuf.at[te], out_hbm.at[s, te, pl.ds(tb0, tb_per_w)], wsem
                )

        def wait_writes(s):
            for te in range(n_et):
                pltpu.make_async_copy(
                    tbuf.at[te], out_hbm.at[s, te, pl.ds(tb0, tb_per_w)], wsem
                ).wait()

        start_idx(0, 0)
        start_idx(1, 1)
        wait_idx(0, 0)
        start_gather(0)
        wait_idx(1, 1)
        start_gather(1)

        def step(s, b, first, last):
            wait_gather(b)
            if not first:
                wait_writes(s - 1)
            transpose(b)
            start_writes(s)
            if not last:
                start_idx(s + 2, b)
                wait_idx(s + 2, b)
                start_gather(b)

        step(0, 0, True, False)
        step(1, 1, False, False)

        @pl.loop(2, seq - 2, step=2)
        def _(s):
            step(s, 0, False, False)
            step(s + 1, 1, False, False)

        step(seq - 2, 0, False, True)
        step(seq - 1, 1, False, True)
        wait_writes(seq - 1)

    return gather_kernel


def kernel(vis_feat, captions, lengths, table):
    batch, seq = captions.shape
    embed_dim = table.shape[1]
    capt = captions.T.reshape(-1)
    gather = _build_gather(batch, seq, embed_dim)
    out5d = gather(table, capt)
    return (
        out5d.transpose(2, 4, 0, 1, 3)
        .reshape(batch, seq, embed_dim // 8, 8)
        .reshape(batch, seq, embed_dim)
    )


# speedup vs baseline: 1.4483x; 1.4483x over previous
import functools

import jax
import jax.numpy as jnp
from jax import lax
from jax.experimental import pallas as pl
from jax.experimental.pallas import tpu as pltpu
from jax.experimental.pallas import tpu_sc as plsc

NUM_CORES = 2
NUM_SUBCORES = 16
NUM_WORKERS = NUM_CORES * NUM_SUBCORES
LANES = 16


@functools.lru_cache(maxsize=None)
def _build_gather(batch, seq, embed_dim):
    assert embed_dim % 8 == 0 and batch % 128 == 0
    n_et = embed_dim // 8
    n_bt = batch // 128
    assert batch % NUM_WORKERS == 0
    bpw = batch // NUM_WORKERS
    assert bpw % 128 == 0
    tb_per_w = bpw // 128
    assert seq % 2 == 0

    mesh = plsc.VectorSubcoreMesh(
        core_axis_name="c",
        subcore_axis_name="s",
        num_cores=NUM_CORES,
        num_subcores=NUM_SUBCORES,
    )

    @functools.partial(
        pl.kernel,
        out_type=jax.ShapeDtypeStruct((seq, n_et, n_bt, 8, 128), jnp.float32),
        mesh=mesh,
        compiler_params=pltpu.CompilerParams(
            use_tc_tiling_on_sc=False, needs_layout_passes=False
        ),
        scratch_types=[
            pltpu.VMEM((bpw, embed_dim), jnp.float32),
            pltpu.VMEM((bpw, embed_dim), jnp.float32),
            pltpu.VMEM((n_et, tb_per_w, 8, 128), jnp.float32),
            pltpu.VMEM((bpw,), jnp.int32),
            pltpu.VMEM((bpw,), jnp.int32),
            pltpu.SemaphoreType.DMA,
            pltpu.SemaphoreType.DMA,
            pltpu.SemaphoreType.DMA,
            pltpu.SemaphoreType.DMA,
            pltpu.SemaphoreType.DMA,
        ],
    )
    def gather_kernel(
        table_hbm, capt_hbm, out_hbm,
        rows0, rows1, tbuf, idx0, idx1,
        g0, g1, i0, i1, wsem,
    ):
        wid = lax.axis_index("s") * NUM_CORES + lax.axis_index("c")
        b0 = wid * bpw
        tb0 = wid * tb_per_w
        rows = (rows0, rows1)
        idxs = (idx0, idx1)
        gsem = (g0, g1)
        isem = (i0, i1)
        iota = lax.iota(jnp.int32, LANES)

        def idx_slice(s):
            return capt_hbm.at[pl.ds(s * batch + b0, bpw)]

        def start_idx(s, b):
            pltpu.async_copy(idx_slice(s), idxs[b], isem[b])

        def wait_idx(s, b):
            pltpu.make_async_copy(idx_slice(s), idxs[b], isem[b]).wait()

        def start_gather(b):
            pltpu.async_copy(table_hbm.at[idxs[b]], rows[b], gsem[b])

        def wait_gather(b):
            pltpu.make_async_copy(table_hbm.at[idxs[b]], rows[b], gsem[b]).wait()

        def transpose(b):
            src = rows[b]

            @pl.loop(0, n_et)
            def _(te):
                col0 = jnp.full((LANES,), te * 8, jnp.int32)
                cols = [col0 + e_in for e_in in range(8)]
                for tb in range(tb_per_w):
                    for bg in range(128 // LANES):
                        rvec = iota + (tb * 128 + bg * LANES)
                        vals = [
                            plsc.load_gather(src, [rvec, cols[e_in]])
                            for e_in in range(8)
                        ]
                        for e_in in range(8):
                            tbuf[te, tb, e_in, pl.ds(bg * LANES, LANES)] = vals[e_in]

        def start_writes(s):
            for te in range(n_et):
                pltpu.async_copy(
                    tbuf.at[te], out_hbm.at[s, te, pl.ds(tb0, tb_per_w)], wsem
                )

        def wait_writes(s):
            for te in range(n_et):
                pltpu.make_async_copy(
                    tbuf.at[te], out_hbm.at[s, te, pl.ds(tb0, tb_per_w)], wsem
                ).wait()

        start_idx(0, 0)
        start_idx(1, 1)
        wait_idx(0, 0)
        start_gather(0)
        wait_idx(1, 1)
        start_gather(1)

        def step(s, b):
            wait_gather(b)

            @pl.when(s >= 1)
            def _():
                wait_writes(s - 1)

            transpose(b)
            start_writes(s)

            @pl.when(s + 2 < seq)
            def _():
                start_idx(s + 2, b)
                wait_idx(s + 2, b)
                start_gather(b)

        @pl.loop(0, seq, step=2)
        def _(s):
            step(s, 0)
            step(s + 1, 1)

        wait_writes(seq - 1)

    return gather_kernel


def kernel(vis_feat, captions, lengths, table):
    batch, seq = captions.shape
    embed_dim = table.shape[1]
    capt = captions.T.reshape(-1)
    gather = _build_gather(batch, seq, embed_dim)
    out5d = gather(table, capt)
    return (
        out5d.transpose(2, 4, 0, 1, 3)
        .reshape(batch, seq, embed_dim // 8, 8)
        .reshape(batch, seq, embed_dim)
    )


# speedup vs baseline: 1.8861x; 1.3023x over previous
import functools

import jax
import jax.numpy as jnp
from jax import lax
from jax.experimental import pallas as pl
from jax.experimental.pallas import tpu as pltpu
from jax.experimental.pallas import tpu_sc as plsc

NUM_CORES = 2
NUM_SUBCORES = 16
NUM_WORKERS = NUM_CORES * NUM_SUBCORES
LANES = 16


@functools.lru_cache(maxsize=None)
def _build_gather(batch, seq, embed_dim):
    assert embed_dim % 8 == 0 and batch % 128 == 0
    n_et = embed_dim // 8
    n_bt = batch // 128
    assert batch % NUM_WORKERS == 0
    bpw = batch // NUM_WORKERS
    assert bpw % 128 == 0
    tb_per_w = bpw // 128
    assert seq % 2 == 0

    mesh = plsc.VectorSubcoreMesh(
        core_axis_name="c",
        subcore_axis_name="s",
        num_cores=NUM_CORES,
        num_subcores=NUM_SUBCORES,
    )

    @functools.partial(
        pl.kernel,
        out_type=jax.ShapeDtypeStruct((seq, n_et, n_bt, 8, 128), jnp.float32),
        mesh=mesh,
        compiler_params=pltpu.CompilerParams(
            use_tc_tiling_on_sc=False, needs_layout_passes=False
        ),
        scratch_types=[
            pltpu.VMEM((bpw, embed_dim), jnp.float32),
            pltpu.VMEM((bpw, embed_dim), jnp.float32),
            pltpu.VMEM((n_et, tb_per_w, 8, 128), jnp.float32),
            pltpu.VMEM((bpw,), jnp.int32),
            pltpu.VMEM((bpw,), jnp.int32),
            pltpu.SemaphoreType.DMA,
            pltpu.SemaphoreType.DMA,
            pltpu.SemaphoreType.DMA,
            pltpu.SemaphoreType.DMA,
            pltpu.SemaphoreType.DMA,
        ],
    )
    def gather_kernel(
        table_hbm, capt_hbm, out_hbm,
        rows0, rows1, tbuf, idx0, idx1,
        g0, g1, i0, i1, wsem,
    ):
        wid = lax.axis_index("s") * NUM_CORES + lax.axis_index("c")
        b0 = wid * bpw
        tb0 = wid * tb_per_w
        rows = (rows0, rows1)
        idxs = (idx0, idx1)
        gsem = (g0, g1)
        isem = (i0, i1)
        iota = lax.iota(jnp.int32, LANES)

        def idx_slice(s):
            return capt_hbm.at[pl.ds(s * batch + b0, bpw)]

        def start_idx(s, b):
            pltpu.async_copy(idx_slice(s), idxs[b], isem[b])

        def wait_idx(s, b):
            pltpu.make_async_copy(idx_slice(s), idxs[b], isem[b]).wait()

        def start_gather(b):
            pltpu.async_copy(table_hbm.at[idxs[b]], rows[b], gsem[b])

        def wait_gather(b):
            pltpu.make_async_copy(table_hbm.at[idxs[b]], rows[b], gsem[b]).wait()

        def transpose(b):
            src = rows[b]
            for eb in range(embed_dim // LANES):
                c0 = eb * LANES

                @pl.loop(0, LANES)
                def _(d):
                    eg = ((iota + d) & (LANES - 1)) + c0
                    tev = eg >> 3
                    einv = eg & 7
                    for tb in range(tb_per_w):
                        tbv = jnp.full((LANES,), tb, jnp.int32)
                        for bg in range(128 // LANES):
                            rowv = iota + (tb * 128 + bg * LANES)
                            bv = iota + (bg * LANES)
                            vals = plsc.load_gather(src, [rowv, eg])
                            plsc.store_scatter(tbuf, [tev, tbv, einv, bv], vals)

        def start_writes(s):
            for te in range(n_et):
                pltpu.async_copy(
                    tbuf.at[te], out_hbm.at[s, te, pl.ds(tb0, tb_per_w)], wsem
                )

        def wait_writes(s):
            for te in range(n_et):
                pltpu.make_async_copy(
                    tbuf.at[te], out_hbm.at[s, te, pl.ds(tb0, tb_per_w)], wsem
                ).wait()

        start_idx(0, 0)
        start_idx(1, 1)
        wait_idx(0, 0)
        start_gather(0)
        wait_idx(1, 1)
        start_gather(1)

        def step(s, b):
            wait_gather(b)

            @pl.when(s >= 1)
            def _():
                wait_writes(s - 1)

            transpose(b)
            start_writes(s)

            @pl.when(s + 2 < seq)
            def _():
                start_idx(s + 2, b)
                wait_idx(s + 2, b)
                start_gather(b)

        @pl.loop(0, seq, step=2)
        def _(s):
            step(s, 0)
            step(s + 1, 1)

        wait_writes(seq - 1)

    return gather_kernel


def kernel(vis_feat, captions, lengths, table):
    batch, seq = captions.shape
    embed_dim = table.shape[1]
    capt = captions.T.reshape(-1)
    gather = _build_gather(batch, seq, embed_dim)
    out5d = gather(table, capt)
    return (
        out5d.transpose(2, 4, 0, 1, 3)
        .reshape(batch, seq, embed_dim // 8, 8)
        .reshape(batch, seq, embed_dim)
    )


# speedup vs baseline: 1.8980x; 1.0063x over previous
import functools

import jax
import jax.numpy as jnp
from jax import lax
from jax.experimental import pallas as pl
from jax.experimental.pallas import tpu as pltpu
from jax.experimental.pallas import tpu_sc as plsc

NUM_CORES = 2
NUM_SUBCORES = 16
NUM_WORKERS = NUM_CORES * NUM_SUBCORES
LANES = 16


@functools.lru_cache(maxsize=None)
def _build_gather(batch, seq, embed_dim):
    assert embed_dim % 8 == 0 and batch % 128 == 0
    n_et = embed_dim // 8
    n_bt = batch // 128
    assert batch % NUM_WORKERS == 0
    bpw = batch // NUM_WORKERS
    assert bpw % 128 == 0
    tb_per_w = bpw // 128
    assert seq % 2 == 0

    mesh = plsc.VectorSubcoreMesh(
        core_axis_name="c",
        subcore_axis_name="s",
        num_cores=NUM_CORES,
        num_subcores=NUM_SUBCORES,
    )

    @functools.partial(
        pl.kernel,
        out_type=jax.ShapeDtypeStruct((seq, n_et, n_bt * 8, 128), jnp.float32),
        mesh=mesh,
        compiler_params=pltpu.CompilerParams(
            use_tc_tiling_on_sc=False, needs_layout_passes=False
        ),
        scratch_types=[
            pltpu.VMEM((bpw, embed_dim), jnp.float32),
            pltpu.VMEM((bpw, embed_dim), jnp.float32),
            pltpu.VMEM((n_et * tb_per_w * 8, 128), jnp.float32),
            pltpu.VMEM((bpw,), jnp.int32),
            pltpu.VMEM((bpw,), jnp.int32),
            pltpu.SemaphoreType.DMA,
            pltpu.SemaphoreType.DMA,
            pltpu.SemaphoreType.DMA,
            pltpu.SemaphoreType.DMA,
            pltpu.SemaphoreType.DMA,
        ],
    )
    def gather_kernel(
        table_hbm, capt_hbm, out_hbm,
        rows0, rows1, tbuf, idx0, idx1,
        g0, g1, i0, i1, wsem,
    ):
        wid = lax.axis_index("s") * NUM_CORES + lax.axis_index("c")
        b0 = wid * bpw
        tb0 = wid * tb_per_w
        rows = (rows0, rows1)
        idxs = (idx0, idx1)
        gsem = (g0, g1)
        isem = (i0, i1)
        iota = lax.iota(jnp.int32, LANES)

        def idx_slice(s):
            return capt_hbm.at[pl.ds(s * batch + b0, bpw)]

        def start_idx(s, b):
            pltpu.async_copy(idx_slice(s), idxs[b], isem[b])

        def wait_idx(s, b):
            pltpu.make_async_copy(idx_slice(s), idxs[b], isem[b]).wait()

        def start_gather(b):
            pltpu.async_copy(table_hbm.at[idxs[b]], rows[b], gsem[b])

        def wait_gather(b):
            pltpu.make_async_copy(table_hbm.at[idxs[b]], rows[b], gsem[b]).wait()

        def transpose(b):
            src = rows[b]
            for eb in range(embed_dim // LANES):
                c0 = eb * LANES

                @pl.loop(0, LANES)
                def _(d):
                    eg = ((iota + d) & (LANES - 1)) + c0
                    rv0 = ((eg >> 3) * (tb_per_w * 8)) + (eg & 7)
                    for tb in range(tb_per_w):
                        rv = rv0 + tb * 8
                        for bg in range(128 // LANES):
                            rowv = iota + (tb * 128 + bg * LANES)
                            bv = iota + (bg * LANES)
                            vals = plsc.load_gather(src, [rowv, eg])
                            plsc.store_scatter(tbuf, [rv, bv], vals)

        def start_writes(s):
            for te in range(n_et):
                pltpu.async_copy(
                    tbuf.at[pl.ds(te * tb_per_w * 8, tb_per_w * 8)],
                    out_hbm.at[s, te, pl.ds(tb0 * 8, tb_per_w * 8)],
                    wsem,
                )

        def wait_writes(s):
            for te in range(n_et):
                pltpu.make_async_copy(
                    tbuf.at[pl.ds(te * tb_per_w * 8, tb_per_w * 8)],
                    out_hbm.at[s, te, pl.ds(tb0 * 8, tb_per_w * 8)],
                    wsem,
                ).wait()

        start_idx(0, 0)
        start_idx(1, 1)
        wait_idx(0, 0)
        start_gather(0)
        wait_idx(1, 1)
        start_gather(1)

        def step(s, b):
            wait_gather(b)

            @pl.when(s >= 1)
            def _():
                wait_writes(s - 1)

            transpose(b)
            start_writes(s)

            @pl.when(s + 2 < seq)
            def _():
                start_idx(s + 2, b)
                wait_idx(s + 2, b)
                start_gather(b)

        @pl.loop(0, seq, step=2)
        def _(s):
            step(s, 0)
            step(s + 1, 1)

        wait_writes(seq - 1)

    return gather_kernel


def kernel(vis_feat, captions, lengths, table):
    batch, seq = captions.shape
    embed_dim = table.shape[1]
    capt = captions.T.reshape(-1)
    gather = _build_gather(batch, seq, embed_dim)
    out4d = gather(table, capt)
    out5d = out4d.reshape(seq, embed_dim // 8, batch // 128, 8, 128)
    return (
        out5d.transpose(2, 4, 0, 1, 3)
        .reshape(batch, seq, embed_dim // 8, 8)
        .reshape(batch, seq, embed_dim)
    )


# speedup vs baseline: 1.9420x; 1.0232x over previous
import functools

import jax
import jax.numpy as jnp
from jax import lax
from jax.experimental import pallas as pl
from jax.experimental.pallas import tpu as pltpu
from jax.experimental.pallas import tpu_sc as plsc

NUM_CORES = 2
NUM_SUBCORES = 16
NUM_WORKERS = NUM_CORES * NUM_SUBCORES
LANES = 16


@functools.lru_cache(maxsize=None)
def _build_gather(batch, seq, embed_dim):
    assert embed_dim % 8 == 0 and batch % 128 == 0
    n_et = embed_dim // 8
    n_bt = batch // 128
    assert batch % NUM_WORKERS == 0
    bpw = batch // NUM_WORKERS
    assert bpw % 128 == 0
    tb_per_w = bpw // 128
    assert seq % 2 == 0

    mesh = plsc.VectorSubcoreMesh(
        core_axis_name="c",
        subcore_axis_name="s",
        num_cores=NUM_CORES,
        num_subcores=NUM_SUBCORES,
    )

    @functools.partial(
        pl.kernel,
        out_type=jax.ShapeDtypeStruct((seq, n_et, n_bt * 8, 128), jnp.float32),
        mesh=mesh,
        compiler_params=pltpu.CompilerParams(
            use_tc_tiling_on_sc=False, needs_layout_passes=False
        ),
        scratch_types=[
            pltpu.VMEM((bpw, embed_dim), jnp.float32),
            pltpu.VMEM((bpw, embed_dim), jnp.float32),
            pltpu.VMEM((2 * tb_per_w * 8, 128), jnp.float32),
            pltpu.VMEM((2 * tb_per_w * 8, 128), jnp.float32),
            pltpu.VMEM((bpw,), jnp.int32),
            pltpu.VMEM((bpw,), jnp.int32),
            pltpu.SemaphoreType.DMA,
            pltpu.SemaphoreType.DMA,
            pltpu.SemaphoreType.DMA,
            pltpu.SemaphoreType.DMA,
            pltpu.SemaphoreType.DMA,
            pltpu.SemaphoreType.DMA,
        ],
    )
    def gather_kernel(
        table_hbm, capt_hbm, out_hbm,
        rows0, rows1, tbuf0, tbuf1, idx0, idx1,
        g0, g1, i0, i1, w0, w1,
    ):
        wid = lax.axis_index("s") * NUM_CORES + lax.axis_index("c")
        b0 = wid * bpw
        tb0 = wid * tb_per_w
        rows = (rows0, rows1)
        idxs = (idx0, idx1)
        gsem = (g0, g1)
        isem = (i0, i1)
        iota = lax.iota(jnp.int32, LANES)

        def idx_slice(s):
            return capt_hbm.at[pl.ds(s * batch + b0, bpw)]

        def start_idx(s, b):
            pltpu.async_copy(idx_slice(s), idxs[b], isem[b])

        def wait_idx(s, b):
            pltpu.make_async_copy(idx_slice(s), idxs[b], isem[b]).wait()

        def start_gather(b):
            pltpu.async_copy(table_hbm.at[idxs[b]], rows[b], gsem[b])

        def wait_gather(b):
            pltpu.make_async_copy(table_hbm.at[idxs[b]], rows[b], gsem[b]).wait()

        tbufs = (tbuf0, tbuf1)
        wsems = (w0, w1)

        def transpose_eb(b, eb, p):
            src = rows[b]
            c0 = eb * LANES

            @pl.loop(0, LANES)
            def _(d):
                eg = ((iota + d) & (LANES - 1)) + c0
                rv0 = (((eg >> 3) & 1) * (tb_per_w * 8)) + (eg & 7)
                for tb in range(tb_per_w):
                    rv = rv0 + tb * 8
                    for bg in range(128 // LANES):
                        rowv = iota + (tb * 128 + bg * LANES)
                        bv = iota + (bg * LANES)
                        vals = plsc.load_gather(src, [rowv, eg])
                        plsc.store_scatter(tbufs[p], [rv, bv], vals)

        def start_writes(s, eb, p):
            for tl in range(2):
                pltpu.async_copy(
                    tbufs[p].at[pl.ds(tl * tb_per_w * 8, tb_per_w * 8)],
                    out_hbm.at[s, 2 * eb + tl, pl.ds(tb0 * 8, tb_per_w * 8)],
                    wsems[p],
                )

        def wait_writes(s, eb, p):
            for tl in range(2):
                pltpu.make_async_copy(
                    tbufs[p].at[pl.ds(tl * tb_per_w * 8, tb_per_w * 8)],
                    out_hbm.at[s, 2 * eb + tl, pl.ds(tb0 * 8, tb_per_w * 8)],
                    wsems[p],
                ).wait()

        start_idx(0, 0)
        start_idx(1, 1)
        wait_idx(0, 0)
        start_gather(0)
        wait_idx(1, 1)
        start_gather(1)

        n_eb = embed_dim // LANES

        def step(s, b):
            wait_gather(b)
            for eb in range(n_eb):
                p = eb % 2
                if eb < 2:
                    @pl.when(s >= 1)
                    def _(eb=eb, p=p):
                        wait_writes(s - 1, eb + 2, p)

                else:
                    wait_writes(s, eb - 2, p)
                transpose_eb(b, eb, p)
                start_writes(s, eb, p)

            @pl.when(s + 2 < seq)
            def _():
                start_idx(s + 2, b)
                wait_idx(s + 2, b)
                start_gather(b)

        @pl.loop(0, seq, step=2)
        def _(s):
            step(s, 0)
            step(s + 1, 1)

        wait_writes(seq - 1, n_eb - 2, 0)
        wait_writes(seq - 1, n_eb - 1, 1)

    return gather_kernel


def kernel(vis_feat, captions, lengths, table):
    batch, seq = captions.shape
    embed_dim = table.shape[1]
    capt = captions.T.reshape(-1)
    gather = _build_gather(batch, seq, embed_dim)
    out4d = gather(table, capt)
    out5d = out4d.reshape(seq, embed_dim // 8, batch // 128, 8, 128)
    return (
        out5d.transpose(2, 4, 0, 1, 3)
        .reshape(batch, seq, embed_dim // 8, 8)
        .reshape(batch, seq, embed_dim)
    )


# speedup vs baseline: 1.9854x; 1.0224x over previous
import functools

import jax
import jax.numpy as jnp
from jax import lax
from jax.experimental import pallas as pl
from jax.experimental.pallas import tpu as pltpu
from jax.experimental.pallas import tpu_sc as plsc

NUM_CORES = 2
NUM_SUBCORES = 16
NUM_WORKERS = NUM_CORES * NUM_SUBCORES
LANES = 16


@functools.lru_cache(maxsize=None)
def _build_gather(batch, seq, embed_dim):
    assert embed_dim % 8 == 0 and batch % 128 == 0
    n_et = embed_dim // 8
    n_bt = batch // 128
    assert batch % NUM_WORKERS == 0
    bpw = batch // NUM_WORKERS
    assert bpw % 128 == 0
    tb_per_w = bpw // 128
    assert seq % 2 == 0

    mesh = plsc.VectorSubcoreMesh(
        core_axis_name="c",
        subcore_axis_name="s",
        num_cores=NUM_CORES,
        num_subcores=NUM_SUBCORES,
    )

    @functools.partial(
        pl.kernel,
        out_type=jax.ShapeDtypeStruct((seq, n_et, n_bt * 8, 128), jnp.float32),
        mesh=mesh,
        compiler_params=pltpu.CompilerParams(
            use_tc_tiling_on_sc=False, needs_layout_passes=False
        ),
        scratch_types=[
            pltpu.VMEM((bpw, embed_dim), jnp.float32),
            pltpu.VMEM((bpw, embed_dim), jnp.float32),
            pltpu.VMEM((2 * tb_per_w * 8, 128), jnp.float32),
            pltpu.VMEM((2 * tb_per_w * 8, 128), jnp.float32),
            pltpu.VMEM((bpw,), jnp.int32),
            pltpu.VMEM((bpw,), jnp.int32),
            pltpu.SemaphoreType.DMA,
            pltpu.SemaphoreType.DMA,
            pltpu.SemaphoreType.DMA,
            pltpu.SemaphoreType.DMA,
            pltpu.SemaphoreType.DMA,
            pltpu.SemaphoreType.DMA,
        ],
    )
    def gather_kernel(
        table_hbm, capt_hbm, out_hbm,
        rows0, rows1, tbuf0, tbuf1, idx0, idx1,
        g0, g1, i0, i1, w0, w1,
    ):
        wid = lax.axis_index("s") * NUM_CORES + lax.axis_index("c")
        b0 = wid * bpw
        tb0 = wid * tb_per_w
        rows = (rows0, rows1)
        idxs = (idx0, idx1)
        gsem = (g0, g1)
        isem = (i0, i1)
        iota = lax.iota(jnp.int32, LANES)

        def idx_slice(s):
            return capt_hbm.at[pl.ds(s * batch + b0, bpw)]

        def start_idx(s, b):
            pltpu.async_copy(idx_slice(s), idxs[b], isem[b])

        def wait_idx(s, b):
            pltpu.make_async_copy(idx_slice(s), idxs[b], isem[b]).wait()

        def start_gather(b):
            pltpu.async_copy(table_hbm.at[idxs[b]], rows[b], gsem[b])

        def wait_gather(b):
            pltpu.make_async_copy(table_hbm.at[idxs[b]], rows[b], gsem[b]).wait()

        tbufs = (tbuf0, tbuf1)
        wsems = (w0, w1)

        def transpose_eb(b, eb, p):
            src = rows[b]
            c0 = eb * LANES

            @pl.loop(0, LANES)
            def _(d):
                eg = ((iota + d) & (LANES - 1)) + c0
                rv0 = (((eg >> 3) & 1) * (tb_per_w * 8)) + (eg & 7)
                for tb in range(tb_per_w):
                    rv = rv0 + tb * 8
                    for bg in range(128 // LANES):
                        rowv = iota + (tb * 128 + bg * LANES)
                        bv = iota + (bg * LANES)
                        vals = plsc.load_gather(src, [rowv, eg])
                        plsc.store_scatter(tbufs[p], [rv, bv], vals)

        def start_writes(s, eb, p):
            for tl in range(2):
                pltpu.async_copy(
                    tbufs[p].at[pl.ds(tl * tb_per_w * 8, tb_per_w * 8)],
                    out_hbm.at[s, 2 * eb + tl, pl.ds(tb0 * 8, tb_per_w * 8)],
                    wsems[p],
                )

        def wait_writes(s, eb, p):
            for tl in range(2):
                pltpu.make_async_copy(
                    tbufs[p].at[pl.ds(tl * tb_per_w * 8, tb_per_w * 8)],
                    out_hbm.at[s, 2 * eb + tl, pl.ds(tb0 * 8, tb_per_w * 8)],
                    wsems[p],
                ).wait()

        start_idx(0, 0)
        start_idx(1, 1)
        wait_idx(0, 0)
        start_gather(0)
        wait_idx(1, 1)
        start_gather(1)

        n_eb = embed_dim // LANES

        def step(s, b):
            wait_gather(b)

            @pl.when(s + 2 < seq)
            def _():
                start_idx(s + 2, b)

            for eb in range(n_eb):
                p = eb % 2
                if eb < 2:
                    @pl.when(s >= 1)
                    def _(eb=eb, p=p):
                        wait_writes(s - 1, eb + 2, p)

                else:
                    wait_writes(s, eb - 2, p)
                transpose_eb(b, eb, p)
                start_writes(s, eb, p)

            @pl.when(s + 2 < seq)
            def _():
                wait_idx(s + 2, b)
                start_gather(b)

        @pl.loop(0, seq, step=2)
        def _(s):
            step(s, 0)
            step(s + 1, 1)

        wait_writes(seq - 1, n_eb - 2, 0)
        wait_writes(seq - 1, n_eb - 1, 1)

    return gather_kernel


def kernel(vis_feat, captions, lengths, table):
    batch, seq = captions.shape
    embed_dim = table.shape[1]
    capt = captions.T.reshape(-1)
    gather = _build_gather(batch, seq, embed_dim)
    out4d = gather(table, capt)
    out5d = out4d.reshape(seq, embed_dim // 8, batch // 128, 8, 128)
    return (
        out5d.transpose(2, 4, 0, 1, 3)
        .reshape(batch, seq, embed_dim // 8, 8)
        .reshape(batch, seq, embed_dim)
    )


# speedup vs baseline: 1.9874x; 1.0010x over previous
import functools

import jax
import jax.numpy as jnp
from jax import lax
from jax.experimental import pallas as pl
from jax.experimental.pallas import tpu as pltpu
from jax.experimental.pallas import tpu_sc as plsc

NUM_CORES = 2
NUM_SUBCORES = 16
NUM_WORKERS = NUM_CORES * NUM_SUBCORES
LANES = 16


@functools.lru_cache(maxsize=None)
def _build_gather(batch, seq, embed_dim):
    assert embed_dim % 8 == 0 and batch % 128 == 0
    n_et = embed_dim // 8
    n_bt = batch // 128
    assert batch % NUM_WORKERS == 0
    bpw = batch // NUM_WORKERS
    assert bpw % 128 == 0
    tb_per_w = bpw // 128
    assert seq % 2 == 0

    mesh = plsc.VectorSubcoreMesh(
        core_axis_name="c",
        subcore_axis_name="s",
        num_cores=NUM_CORES,
        num_subcores=NUM_SUBCORES,
    )

    @functools.partial(
        pl.kernel,
        out_type=jax.ShapeDtypeStruct((seq, n_et, n_bt * 8, 128), jnp.float32),
        mesh=mesh,
        compiler_params=pltpu.CompilerParams(
            use_tc_tiling_on_sc=False, needs_layout_passes=False
        ),
        scratch_types=[
            pltpu.VMEM((bpw, embed_dim), jnp.float32),
            pltpu.VMEM((bpw, embed_dim), jnp.float32),
            pltpu.VMEM((2 * tb_per_w * 8, 128), jnp.float32),
            pltpu.VMEM((2 * tb_per_w * 8, 128), jnp.float32),
            pltpu.VMEM((bpw,), jnp.int32),
            pltpu.VMEM((bpw,), jnp.int32),
            pltpu.SemaphoreType.DMA,
            pltpu.SemaphoreType.DMA,
            pltpu.SemaphoreType.DMA,
            pltpu.SemaphoreType.DMA,
            pltpu.SemaphoreType.DMA,
            pltpu.SemaphoreType.DMA,
        ],
    )
    def gather_kernel(
        table_hbm, capt_hbm, out_hbm,
        rows0, rows1, tbuf0, tbuf1, idx0, idx1,
        g0, g1, i0, i1, w0, w1,
    ):
        wid = lax.axis_index("s") * NUM_CORES + lax.axis_index("c")
        b0 = wid * bpw
        tb0 = wid * tb_per_w
        rows = (rows0, rows1)
        idxs = (idx0, idx1)
        gsem = (g0, g1)
        isem = (i0, i1)
        iota = lax.iota(jnp.int32, LANES)

        def idx_slice(s):
            return capt_hbm.at[pl.ds(s * batch + b0, bpw)]

        def start_idx(s, b):
            pltpu.async_copy(idx_slice(s), idxs[b], isem[b])

        def wait_idx(s, b):
            pltpu.make_async_copy(idx_slice(s), idxs[b], isem[b]).wait()

        def start_gather(b):
            pltpu.async_copy(table_hbm.at[idxs[b]], rows[b], gsem[b])

        def wait_gather(b):
            pltpu.make_async_copy(table_hbm.at[idxs[b]], rows[b], gsem[b]).wait()

        tbufs = (tbuf0, tbuf1)
        wsems = (w0, w1)

        def transpose_eb(b, eb, p):
            src = rows[b]
            c0 = eb * LANES

            @pl.loop(0, LANES)
            def _(d):
                eg = ((iota + d) & (LANES - 1)) + c0
                rv0 = (((eg >> 3) & 1) * (tb_per_w * 8)) + (eg & 7)
                rvs = [rv0 + tb * 8 for tb in range(tb_per_w)]
                for bg in range(128 // LANES):
                    bv = iota + (bg * LANES)
                    for tb in range(tb_per_w):
                        rowv = bv + tb * 128
                        vals = plsc.load_gather(src, [rowv, eg])
                        plsc.store_scatter(tbufs[p], [rvs[tb], bv], vals)

        def start_writes(s, eb, p):
            for tl in range(2):
                pltpu.async_copy(
                    tbufs[p].at[pl.ds(tl * tb_per_w * 8, tb_per_w * 8)],
                    out_hbm.at[s, 2 * eb + tl, pl.ds(tb0 * 8, tb_per_w * 8)],
                    wsems[p],
                )

        def wait_writes(s, eb, p):
            for tl in range(2):
                pltpu.make_async_copy(
                    tbufs[p].at[pl.ds(tl * tb_per_w * 8, tb_per_w * 8)],
                    out_hbm.at[s, 2 * eb + tl, pl.ds(tb0 * 8, tb_per_w * 8)],
                    wsems[p],
                ).wait()

        start_idx(0, 0)
        start_idx(1, 1)
        wait_idx(0, 0)
        start_gather(0)
        wait_idx(1, 1)
        start_gather(1)

        n_eb = embed_dim // LANES

        def step(s, b):
            wait_gather(b)

            @pl.when(s + 2 < seq)
            def _():
                start_idx(s + 2, b)

            for eb in range(n_eb):
                p = eb % 2
                if eb < 2:
                    @pl.when(s >= 1)
                    def _(eb=eb, p=p):
                        wait_writes(s - 1, eb + 2, p)

                else:
                    wait_writes(s, eb - 2, p)
                transpose_eb(b, eb, p)
                start_writes(s, eb, p)

            @pl.when(s + 2 < seq)
            def _():
                wait_idx(s + 2, b)
                start_gather(b)

        @pl.loop(0, seq, step=2)
        def _(s):
            step(s, 0)
            step(s + 1, 1)

        wait_writes(seq - 1, n_eb - 2, 0)
        wait_writes(seq - 1, n_eb - 1, 1)

    return gather_kernel


def kernel(vis_feat, captions, lengths, table):
    batch, seq = captions.shape
    embed_dim = table.shape[1]
    capt = captions.T.reshape(-1)
    gather = _build_gather(batch, seq, embed_dim)
    out4d = gather(table, capt)
    out5d = out4d.reshape(seq, embed_dim // 8, batch // 128, 8, 128)
    return (
        out5d.transpose(2, 4, 0, 1, 3)
        .reshape(batch, seq, embed_dim // 8, 8)
        .reshape(batch, seq, embed_dim)
    )
